# baseline (speedup 1.0000x reference)
import functools

import jax
import jax.numpy as jnp
from jax import lax
from jax.experimental import pallas as pl
from jax.experimental.pallas import tpu as pltpu
from jax.experimental.pallas import tpu_sc as plsc

_N_ENT = 1000000
_N_REL = 1000
_D = 64
_B = 16384
_L = 16
_NW = 32
_BPW = _B // _NW
_C = 128
_NCHUNK = _BPW // _C
_NG = _C // _L
_EB = 2048


def _combine_entities_body(ert_ref, eit_ref, c_ref):
    c_ref[:, 0:_D] = jnp.transpose(ert_ref[...])
    c_ref[:, _D:2 * _D] = jnp.transpose(eit_ref[...])


_combine_entities = pl.pallas_call(
    _combine_entities_body,
    grid=(pl.cdiv(_N_ENT, _EB),),
    in_specs=[
        pl.BlockSpec((_D, _EB), lambda g: (0, g)),
        pl.BlockSpec((_D, _EB), lambda g: (0, g)),
    ],
    out_specs=pl.BlockSpec((_EB, 2 * _D), lambda g: (g, 0)),
    out_shape=jax.ShapeDtypeStruct((_N_ENT, 2 * _D), jnp.float32),
)


def _combine_relations_body(rrt_ref, rit_ref, s_ref):
    s_ref[...] = jnp.zeros_like(s_ref)
    s_ref[:, 0:_D] = jnp.transpose(rrt_ref[...] + rit_ref[...])


_combine_relations = pl.pallas_call(
    _combine_relations_body,
    in_specs=[
        pl.BlockSpec((_D, _N_REL), lambda: (0, 0)),
        pl.BlockSpec((_D, _N_REL), lambda: (0, 0)),
    ],
    out_specs=pl.BlockSpec((_N_REL, 2 * _D), lambda: (0, 0)),
    out_shape=jax.ShapeDtypeStruct((_N_REL, 2 * _D), jnp.float32),
)


def _make_score():
    mesh = plsc.VectorSubcoreMesh(core_axis_name="c", subcore_axis_name="s")

    @functools.partial(
        pl.kernel,
        mesh=mesh,
        out_type=jax.ShapeDtypeStruct((_B,), jnp.float32),
        scratch_types=[
            pltpu.VMEM((_C,), jnp.int32),
            pltpu.VMEM((_C,), jnp.int32),
            pltpu.VMEM((_C,), jnp.int32),
            pltpu.VMEM((_C, 2 * _D), jnp.float32),
            pltpu.VMEM((_C, 2 * _D), jnp.float32),
            pltpu.VMEM((_C, 2 * _D), jnp.float32),
            pltpu.VMEM((_BPW,), jnp.float32),
            pltpu.SemaphoreType.DMA,
        ],
        compiler_params=pltpu.CompilerParams(use_tc_tiling_on_sc=True),
    )
    def score(heads_hbm, rels_hbm, tails_hbm, c_hbm, s_hbm, out_hbm,
              idx_h, idx_t, idx_r, h_v, t_v, s_v, out_v, sem):
        wid = lax.axis_index("s") * 2 + lax.axis_index("c")
        base = pl.multiple_of(wid * _BPW, _BPW)
        lanes = lax.iota(jnp.int32, _L)
        perms = [lanes ^ sh for sh in (8, 4, 2, 1)]
        lane_masks = [lanes == k for k in range(_L)]

        def chunk_body(c, carry):
            cbase = base + c * _C
            pltpu.sync_copy(heads_hbm.at[pl.ds(cbase, _C)], idx_h)
            pltpu.sync_copy(tails_hbm.at[pl.ds(cbase, _C)], idx_t)
            pltpu.sync_copy(rels_hbm.at[pl.ds(cbase, _C)], idx_r)

            cps = [
                pltpu.async_copy(c_hbm.at[idx_h], h_v, sem),
                pltpu.async_copy(c_hbm.at[idx_t], t_v, sem),
                pltpu.async_copy(s_hbm.at[idx_r], s_v, sem),
            ]
            for cp in cps:
                cp.wait()

            def group_body(g, carry2):
                out_vec = jnp.zeros((_L,), jnp.float32)
                for k in range(_L):
                    i = g * _L + k
                    acc = jnp.zeros((_L,), jnp.float32)
                    for j in range(_D // _L):
                        sl = pl.ds(j * _L, _L)
                        sl2 = pl.ds(_D + j * _L, _L)
                        hr = h_v[i, sl]
                        hi = h_v[i, sl2]
                        tr = t_v[i, sl]
                        ti = t_v[i, sl2]
                        s = s_v[i, sl]
                        acc = acc + s * ((hr - hi) * tr + (hr + hi) * ti)
                    for perm in perms:
                        acc = acc + _lane_shuffle(acc, perm)
                    out_vec = lax.select(lane_masks[k], acc, out_vec)
                out_v[pl.ds(c * _C + g * _L, _L)] = out_vec
                return carry2

            lax.fori_loop(0, _NG, group_body, 0)
            return carry

        lax.fori_loop(0, _NCHUNK, chunk_body, 0)
        pltpu.sync_copy(out_v, out_hbm.at[pl.ds(base, _BPW)])

    return score


_GATHER_DNUMS = lax.GatherDimensionNumbers(
    offset_dims=(), collapsed_slice_dims=(0,), start_index_map=(0,))


def _lane_shuffle(v, perm):
    return lax.gather(v, perm[:, None], _GATHER_DNUMS, slice_sizes=(1,),
                      mode=lax.GatherScatterMode.PROMISE_IN_BOUNDS)


_score = _make_score()


def kernel(heads, relations, tails, entity_real, entity_imag,
           relation_real, relation_imag):
    comb = jnp.concatenate([entity_real, entity_imag], axis=1)
    srel = _combine_relations(relation_real.T, relation_imag.T)
    return _score(heads.astype(jnp.int32), relations.astype(jnp.int32),
                  tails.astype(jnp.int32), comb, srel)


# speedup vs baseline: 1.6758x; 1.6758x over previous
import functools

import jax
import jax.numpy as jnp
from jax import lax
from jax.experimental import pallas as pl
from jax.experimental.pallas import tpu as pltpu
from jax.experimental.pallas import tpu_sc as plsc

_N_ENT = 1000000
_N_REL = 1000
_D = 64
_B = 16384
_L = 16
_NW = 32
_BPW = _B // _NW
_C = 128
_NCHUNK = _BPW // _C
_NG = _C // _L
_EB = 4096


def _combine_entities_body(ert_ref, eit_ref, c_ref):
    c_ref[:, 0:_D] = jnp.transpose(ert_ref[...])
    c_ref[:, _D:2 * _D] = jnp.transpose(eit_ref[...])


_combine_entities = pl.pallas_call(
    _combine_entities_body,
    grid=(pl.cdiv(_N_ENT, _EB),),
    in_specs=[
        pl.BlockSpec((_D, _EB), lambda g: (0, g)),
        pl.BlockSpec((_D, _EB), lambda g: (0, g)),
    ],
    out_specs=pl.BlockSpec((_EB, 2 * _D), lambda g: (g, 0)),
    out_shape=jax.ShapeDtypeStruct((_N_ENT, 2 * _D), jnp.float32),
)


def _combine_relations_body(rrt_ref, rit_ref, s_ref):
    s_ref[...] = jnp.zeros_like(s_ref)
    s_ref[:, 0:_D] = jnp.transpose(rrt_ref[...] + rit_ref[...])


_combine_relations = pl.pallas_call(
    _combine_relations_body,
    in_specs=[
        pl.BlockSpec((_D, _N_REL), lambda: (0, 0)),
        pl.BlockSpec((_D, _N_REL), lambda: (0, 0)),
    ],
    out_specs=pl.BlockSpec((_N_REL, 2 * _D), lambda: (0, 0)),
    out_shape=jax.ShapeDtypeStruct((_N_REL, 2 * _D), jnp.float32),
)


def _make_score():
    mesh = plsc.VectorSubcoreMesh(core_axis_name="c", subcore_axis_name="s")

    @functools.partial(
        pl.kernel,
        mesh=mesh,
        out_type=jax.ShapeDtypeStruct((_B,), jnp.float32),
        scratch_types=[
            pltpu.VMEM((_C,), jnp.int32),
            pltpu.VMEM((_C,), jnp.int32),
            pltpu.VMEM((_C,), jnp.int32),
            pltpu.VMEM((_C, 2 * _D), jnp.float32),
            pltpu.VMEM((_C, 2 * _D), jnp.float32),
            pltpu.VMEM((_C, 2 * _D), jnp.float32),
            pltpu.VMEM((_BPW,), jnp.float32),
            pltpu.SemaphoreType.DMA,
        ],
        compiler_params=pltpu.CompilerParams(use_tc_tiling_on_sc=True),
    )
    def score(heads_hbm, rels_hbm, tails_hbm, c_hbm, s_hbm, out_hbm,
              idx_h, idx_t, idx_r, h_v, t_v, s_v, out_v, sem):
        wid = lax.axis_index("s") * 2 + lax.axis_index("c")
        base = pl.multiple_of(wid * _BPW, _BPW)
        lanes = lax.iota(jnp.int32, _L)
        perms = [lanes ^ sh for sh in (8, 4, 2, 1)]
        lane_masks = [lanes == k for k in range(_L)]

        def chunk_body(c, carry):
            cbase = base + c * _C
            pltpu.sync_copy(heads_hbm.at[pl.ds(cbase, _C)], idx_h)
            pltpu.sync_copy(tails_hbm.at[pl.ds(cbase, _C)], idx_t)
            pltpu.sync_copy(rels_hbm.at[pl.ds(cbase, _C)], idx_r)

            cps = [
                pltpu.async_copy(c_hbm.at[idx_h], h_v, sem),
                pltpu.async_copy(c_hbm.at[idx_t], t_v, sem),
                pltpu.async_copy(s_hbm.at[idx_r], s_v, sem),
            ]
            for cp in cps:
                cp.wait()

            def group_body(g, carry2):
                out_vec = jnp.zeros((_L,), jnp.float32)
                for k in range(_L):
                    i = g * _L + k
                    acc = jnp.zeros((_L,), jnp.float32)
                    for j in range(_D // _L):
                        sl = pl.ds(j * _L, _L)
                        sl2 = pl.ds(_D + j * _L, _L)
                        hr = h_v[i, sl]
                        hi = h_v[i, sl2]
                        tr = t_v[i, sl]
                        ti = t_v[i, sl2]
                        s = s_v[i, sl]
                        acc = acc + s * ((hr - hi) * tr + (hr + hi) * ti)
                    for perm in perms:
                        acc = acc + _lane_shuffle(acc, perm)
                    out_vec = lax.select(lane_masks[k], acc, out_vec)
                out_v[pl.ds(c * _C + g * _L, _L)] = out_vec
                return carry2

            lax.fori_loop(0, _NG, group_body, 0)
            return carry

        lax.fori_loop(0, _NCHUNK, chunk_body, 0)
        pltpu.sync_copy(out_v, out_hbm.at[pl.ds(base, _BPW)])

    return score


_GATHER_DNUMS = lax.GatherDimensionNumbers(
    offset_dims=(), collapsed_slice_dims=(0,), start_index_map=(0,))


def _lane_shuffle(v, perm):
    return lax.gather(v, perm[:, None], _GATHER_DNUMS, slice_sizes=(1,),
                      mode=lax.GatherScatterMode.PROMISE_IN_BOUNDS)


_score = _make_score()


def kernel(heads, relations, tails, entity_real, entity_imag,
           relation_real, relation_imag):
    comb = _combine_entities(entity_real.T, entity_imag.T)
    srel = _combine_relations(relation_real.T, relation_imag.T)
    return _score(heads.astype(jnp.int32), relations.astype(jnp.int32),
                  tails.astype(jnp.int32), comb, srel)


# speedup vs baseline: 1.9071x; 1.1380x over previous
import functools

import jax
import jax.numpy as jnp
from jax import lax
from jax.experimental import pallas as pl
from jax.experimental.pallas import tpu as pltpu
from jax.experimental.pallas import tpu_sc as plsc

_N_ENT = 1000000
_N_REL = 1000
_D = 64
_B = 16384
_L = 16
_NW = 32
_BPW = _B // _NW
_C = 128
_NCHUNK = _BPW // _C
_NG = _C // _L
_EB = 8192


def _combine_entities_body(ert_ref, eit_ref, c_ref):
    c_ref[:, 0:_D] = jnp.transpose(ert_ref[...])
    c_ref[:, _D:2 * _D] = jnp.transpose(eit_ref[...])


_combine_entities = pl.pallas_call(
    _combine_entities_body,
    grid=(pl.cdiv(_N_ENT, _EB),),
    in_specs=[
        pl.BlockSpec((_D, _EB), lambda g: (0, g)),
        pl.BlockSpec((_D, _EB), lambda g: (0, g)),
    ],
    out_specs=pl.BlockSpec((_EB, 2 * _D), lambda g: (g, 0)),
    out_shape=jax.ShapeDtypeStruct((_N_ENT, 2 * _D), jnp.float32),
)


def _combine_relations_body(rrt_ref, rit_ref, s_ref):
    s_ref[...] = jnp.zeros_like(s_ref)
    s_ref[:, 0:_D] = jnp.transpose(rrt_ref[...] + rit_ref[...])


_combine_relations = pl.pallas_call(
    _combine_relations_body,
    in_specs=[
        pl.BlockSpec((_D, _N_REL), lambda: (0, 0)),
        pl.BlockSpec((_D, _N_REL), lambda: (0, 0)),
    ],
    out_specs=pl.BlockSpec((_N_REL, 2 * _D), lambda: (0, 0)),
    out_shape=jax.ShapeDtypeStruct((_N_REL, 2 * _D), jnp.float32),
)


def _make_score():
    mesh = plsc.VectorSubcoreMesh(core_axis_name="c", subcore_axis_name="s")

    @functools.partial(
        pl.kernel,
        mesh=mesh,
        out_type=jax.ShapeDtypeStruct((_B,), jnp.float32),
        scratch_types=[
            pltpu.VMEM((_C,), jnp.int32),
            pltpu.VMEM((_C,), jnp.int32),
            pltpu.VMEM((_C,), jnp.int32),
            pltpu.VMEM((_C, 2 * _D), jnp.float32),
            pltpu.VMEM((_C, 2 * _D), jnp.float32),
            pltpu.VMEM((_C, 2 * _D), jnp.float32),
            pltpu.VMEM((_BPW,), jnp.float32),
            pltpu.SemaphoreType.DMA,
        ],
        compiler_params=pltpu.CompilerParams(use_tc_tiling_on_sc=True),
    )
    def score(heads_hbm, rels_hbm, tails_hbm, c_hbm, s_hbm, out_hbm,
              idx_h, idx_t, idx_r, h_v, t_v, s_v, out_v, sem):
        wid = lax.axis_index("s") * 2 + lax.axis_index("c")
        base = pl.multiple_of(wid * _BPW, _BPW)
        lanes = lax.iota(jnp.int32, _L)
        perms = [lanes ^ sh for sh in (8, 4, 2, 1)]
        lane_masks = [lanes == k for k in range(_L)]

        def chunk_body(c, carry):
            cbase = base + c * _C
            pltpu.sync_copy(heads_hbm.at[pl.ds(cbase, _C)], idx_h)
            pltpu.sync_copy(tails_hbm.at[pl.ds(cbase, _C)], idx_t)
            pltpu.sync_copy(rels_hbm.at[pl.ds(cbase, _C)], idx_r)

            cps = [
                pltpu.async_copy(c_hbm.at[idx_h], h_v, sem),
                pltpu.async_copy(c_hbm.at[idx_t], t_v, sem),
                pltpu.async_copy(s_hbm.at[idx_r], s_v, sem),
            ]
            for cp in cps:
                cp.wait()

            def group_body(g, carry2):
                out_vec = jnp.zeros((_L,), jnp.float32)
                for k in range(_L):
                    i = g * _L + k
                    acc = jnp.zeros((_L,), jnp.float32)
                    for j in range(_D // _L):
                        sl = pl.ds(j * _L, _L)
                        sl2 = pl.ds(_D + j * _L, _L)
                        hr = h_v[i, sl]
                        hi = h_v[i, sl2]
                        tr = t_v[i, sl]
                        ti = t_v[i, sl2]
                        s = s_v[i, sl]
                        acc = acc + s * ((hr - hi) * tr + (hr + hi) * ti)
                    for perm in perms:
                        acc = acc + _lane_shuffle(acc, perm)
                    out_vec = lax.select(lane_masks[k], acc, out_vec)
                out_v[pl.ds(c * _C + g * _L, _L)] = out_vec
                return carry2

            lax.fori_loop(0, _NG, group_body, 0)
            return carry

        lax.fori_loop(0, _NCHUNK, chunk_body, 0)
        pltpu.sync_copy(out_v, out_hbm.at[pl.ds(base, _BPW)])

    return score


_GATHER_DNUMS = lax.GatherDimensionNumbers(
    offset_dims=(), collapsed_slice_dims=(0,), start_index_map=(0,))


def _lane_shuffle(v, perm):
    return lax.gather(v, perm[:, None], _GATHER_DNUMS, slice_sizes=(1,),
                      mode=lax.GatherScatterMode.PROMISE_IN_BOUNDS)


_score = _make_score()


def kernel(heads, relations, tails, entity_real, entity_imag,
           relation_real, relation_imag):
    comb = _combine_entities(entity_real.T, entity_imag.T)
    srel = _combine_relations(relation_real.T, relation_imag.T)
    return _score(heads.astype(jnp.int32), relations.astype(jnp.int32),
                  tails.astype(jnp.int32), comb, srel)


# speedup vs baseline: 2.0300x; 1.0645x over previous
import functools

import jax
import jax.numpy as jnp
from jax import lax
from jax.experimental import pallas as pl
from jax.experimental.pallas import tpu as pltpu
from jax.experimental.pallas import tpu_sc as plsc

_N_ENT = 1000000
_N_REL = 1000
_D = 64
_B = 16384
_L = 16
_NW = 32
_BPW = _B // _NW
_C = 128
_NCHUNK = _BPW // _C
_NG = _C // _L
_EB = 16384


def _combine_entities_body(ert_ref, eit_ref, c_ref):
    c_ref[:, 0:_D] = jnp.transpose(ert_ref[...])
    c_ref[:, _D:2 * _D] = jnp.transpose(eit_ref[...])


_combine_entities = pl.pallas_call(
    _combine_entities_body,
    grid=(pl.cdiv(_N_ENT, _EB),),
    in_specs=[
        pl.BlockSpec((_D, _EB), lambda g: (0, g)),
        pl.BlockSpec((_D, _EB), lambda g: (0, g)),
    ],
    out_specs=pl.BlockSpec((_EB, 2 * _D), lambda g: (g, 0)),
    out_shape=jax.ShapeDtypeStruct((_N_ENT, 2 * _D), jnp.float32),
)


def _combine_relations_body(rrt_ref, rit_ref, s_ref):
    s_ref[...] = jnp.zeros_like(s_ref)
    s_ref[:, 0:_D] = jnp.transpose(rrt_ref[...] + rit_ref[...])


_combine_relations = pl.pallas_call(
    _combine_relations_body,
    in_specs=[
        pl.BlockSpec((_D, _N_REL), lambda: (0, 0)),
        pl.BlockSpec((_D, _N_REL), lambda: (0, 0)),
    ],
    out_specs=pl.BlockSpec((_N_REL, 2 * _D), lambda: (0, 0)),
    out_shape=jax.ShapeDtypeStruct((_N_REL, 2 * _D), jnp.float32),
)


def _make_score():
    mesh = plsc.VectorSubcoreMesh(core_axis_name="c", subcore_axis_name="s")

    @functools.partial(
        pl.kernel,
        mesh=mesh,
        out_type=jax.ShapeDtypeStruct((_B,), jnp.float32),
        scratch_types=[
            pltpu.VMEM((_C,), jnp.int32),
            pltpu.VMEM((_C,), jnp.int32),
            pltpu.VMEM((_C,), jnp.int32),
            pltpu.VMEM((_C, 2 * _D), jnp.float32),
            pltpu.VMEM((_C, 2 * _D), jnp.float32),
            pltpu.VMEM((_C, 2 * _D), jnp.float32),
            pltpu.VMEM((_BPW,), jnp.float32),
            pltpu.SemaphoreType.DMA,
        ],
        compiler_params=pltpu.CompilerParams(use_tc_tiling_on_sc=True),
    )
    def score(heads_hbm, rels_hbm, tails_hbm, c_hbm, s_hbm, out_hbm,
              idx_h, idx_t, idx_r, h_v, t_v, s_v, out_v, sem):
        wid = lax.axis_index("s") * 2 + lax.axis_index("c")
        base = pl.multiple_of(wid * _BPW, _BPW)
        lanes = lax.iota(jnp.int32, _L)
        perms = [lanes ^ sh for sh in (8, 4, 2, 1)]
        lane_masks = [lanes == k for k in range(_L)]

        def chunk_body(c, carry):
            cbase = base + c * _C
            pltpu.sync_copy(heads_hbm.at[pl.ds(cbase, _C)], idx_h)
            pltpu.sync_copy(tails_hbm.at[pl.ds(cbase, _C)], idx_t)
            pltpu.sync_copy(rels_hbm.at[pl.ds(cbase, _C)], idx_r)

            cps = [
                pltpu.async_copy(c_hbm.at[idx_h], h_v, sem),
                pltpu.async_copy(c_hbm.at[idx_t], t_v, sem),
                pltpu.async_copy(s_hbm.at[idx_r], s_v, sem),
            ]
            for cp in cps:
                cp.wait()

            def group_body(g, carry2):
                out_vec = jnp.zeros((_L,), jnp.float32)
                for k in range(_L):
                    i = g * _L + k
                    acc = jnp.zeros((_L,), jnp.float32)
                    for j in range(_D // _L):
                        sl = pl.ds(j * _L, _L)
                        sl2 = pl.ds(_D + j * _L, _L)
                        hr = h_v[i, sl]
                        hi = h_v[i, sl2]
                        tr = t_v[i, sl]
                        ti = t_v[i, sl2]
                        s = s_v[i, sl]
                        acc = acc + s * ((hr - hi) * tr + (hr + hi) * ti)
                    for perm in perms:
                        acc = acc + _lane_shuffle(acc, perm)
                    out_vec = lax.select(lane_masks[k], acc, out_vec)
                out_v[pl.ds(c * _C + g * _L, _L)] = out_vec
                return carry2

            lax.fori_loop(0, _NG, group_body, 0)
            return carry

        lax.fori_loop(0, _NCHUNK, chunk_body, 0)
        pltpu.sync_copy(out_v, out_hbm.at[pl.ds(base, _BPW)])

    return score


_GATHER_DNUMS = lax.GatherDimensionNumbers(
    offset_dims=(), collapsed_slice_dims=(0,), start_index_map=(0,))


def _lane_shuffle(v, perm):
    return lax.gather(v, perm[:, None], _GATHER_DNUMS, slice_sizes=(1,),
                      mode=lax.GatherScatterMode.PROMISE_IN_BOUNDS)


_score = _make_score()


def kernel(heads, relations, tails, entity_real, entity_imag,
           relation_real, relation_imag):
    comb = _combine_entities(entity_real.T, entity_imag.T)
    srel = _combine_relations(relation_real.T, relation_imag.T)
    return _score(heads.astype(jnp.int32), relations.astype(jnp.int32),
                  tails.astype(jnp.int32), comb, srel)
